# baseline (speedup 1.0000x reference)
import jax
import jax.numpy as jnp
from jax.experimental import pallas as pl
from jax.experimental.pallas import tpu as pltpu

N = 10000
IN_CH = 512
NUM_RES = 4
HID = 1024
BN = 1000
NSTRIPE = IN_CH // 128
ROWS = NUM_RES * NSTRIPE


def _head_kernel(x_ref, wl_ref, W1_ref, b1_ref, w2_ref, bias_ref, out_ref):
    xlast = x_ref[pl.ds(NUM_RES - 1, NSTRIPE * BN, NUM_RES), :]
    xlast = xlast.reshape(BN, IN_CH)
    h = jnp.dot(xlast.astype(jnp.bfloat16), W1_ref[...],
                preferred_element_type=jnp.float32)
    h = h + b1_ref[...]
    h = h * jax.nn.sigmoid(h)
    acc = jnp.dot(h, w2_ref[...],
                  preferred_element_type=jnp.float32)
    for r in range(NUM_RES - 1):
        xr = x_ref[pl.ds(r, NSTRIPE * BN, NUM_RES), :]
        yr = jnp.dot(xr.reshape(BN, IN_CH), wl_ref[...],
                     preferred_element_type=jnp.float32)
        acc = acc + yr[:, r:r + 1]
    out_ref[...] = acc + bias_ref[...]


def kernel(embedding_0, W_lin, b_lin, W1, b1, W2, b2):
    x = jnp.squeeze(embedding_0, 2)
    x = x.reshape(N, NSTRIPE, 128, NUM_RES)
    x = jnp.transpose(x, (0, 1, 3, 2))
    x = x.reshape(N * ROWS, 128)
    wl = jnp.concatenate(
        [W_lin[:, :, 0], jnp.zeros((1, IN_CH), jnp.float32)], axis=0).T
    bias = (jnp.sum(b_lin) + b2[0]).reshape(1, 1)

    out = pl.pallas_call(
        _head_kernel,
        grid=(N // BN,),
        in_specs=[
            pl.BlockSpec((ROWS * BN, 128), lambda i: (i, 0)),
            pl.BlockSpec((IN_CH, NUM_RES), lambda i: (0, 0)),
            pl.BlockSpec((IN_CH, HID), lambda i: (0, 0)),
            pl.BlockSpec((1, HID), lambda i: (0, 0)),
            pl.BlockSpec((HID, 1), lambda i: (0, 0)),
            pl.BlockSpec((1, 1), lambda i: (0, 0)),
        ],
        out_specs=pl.BlockSpec((BN, 1), lambda i: (i, 0)),
        out_shape=jax.ShapeDtypeStruct((N, 1), jnp.float32),
        compiler_params=pltpu.CompilerParams(dimension_semantics=("parallel",)),
    )(x, wl, W1.astype(jnp.bfloat16), b1.reshape(1, HID),
      W2.reshape(HID, 1), bias)
    return out.reshape(N)


# speedup vs baseline: 1.0007x; 1.0007x over previous
import jax
import jax.numpy as jnp
from jax.experimental import pallas as pl
from jax.experimental.pallas import tpu as pltpu

N = 10000
IN_CH = 512
NUM_RES = 4
HID = 1024
BN = 1000
NSTRIPE = IN_CH // 128
ROWS = NUM_RES * NSTRIPE


def _head_kernel(x_ref, wl_ref, W1_ref, b1_ref, w2_ref, bias_ref, out_ref):
    flat = x_ref.reshape(ROWS * BN, 128)
    xlast = flat[pl.ds(NUM_RES - 1, NSTRIPE * BN, NUM_RES), :]
    xlast = xlast.reshape(BN, IN_CH)
    h = jnp.dot(xlast.astype(jnp.bfloat16), W1_ref[...],
                preferred_element_type=jnp.float32)
    h = h + b1_ref[...]
    h = h * jax.nn.sigmoid(h)
    acc = jnp.dot(h, w2_ref[...],
                  preferred_element_type=jnp.float32)
    for r in range(NUM_RES - 1):
        xr = flat[pl.ds(r, NSTRIPE * BN, NUM_RES), :]
        yr = jnp.dot(xr.reshape(BN, IN_CH), wl_ref[...],
                     preferred_element_type=jnp.float32)
        acc = acc + yr[:, r:r + 1]
    out_ref[...] = acc + bias_ref[...]


def kernel(embedding_0, W_lin, b_lin, W1, b1, W2, b2):
    x = jnp.squeeze(embedding_0, 2)
    x = x.reshape(N, NSTRIPE, 128, NUM_RES)
    x = jnp.transpose(x, (0, 1, 3, 2))
    wl = jnp.concatenate(
        [W_lin[:, :, 0], jnp.zeros((1, IN_CH), jnp.float32)], axis=0).T
    bias = (jnp.sum(b_lin) + b2[0]).reshape(1, 1)

    out = pl.pallas_call(
        _head_kernel,
        grid=(N // BN,),
        in_specs=[
            pl.BlockSpec((BN, NSTRIPE, NUM_RES, 128), lambda i: (i, 0, 0, 0)),
            pl.BlockSpec((IN_CH, NUM_RES), lambda i: (0, 0)),
            pl.BlockSpec((IN_CH, HID), lambda i: (0, 0)),
            pl.BlockSpec((1, HID), lambda i: (0, 0)),
            pl.BlockSpec((HID, 1), lambda i: (0, 0)),
            pl.BlockSpec((1, 1), lambda i: (0, 0)),
        ],
        out_specs=pl.BlockSpec((BN, 1), lambda i: (i, 0)),
        out_shape=jax.ShapeDtypeStruct((N, 1), jnp.float32),
        compiler_params=pltpu.CompilerParams(dimension_semantics=("parallel",)),
    )(x, wl, W1.astype(jnp.bfloat16), b1.reshape(1, HID),
      W2.reshape(HID, 1), bias)
    return out.reshape(N)


# speedup vs baseline: 1.0339x; 1.0332x over previous
import jax
import jax.numpy as jnp
from jax.experimental import pallas as pl
from jax.experimental.pallas import tpu as pltpu

N = 10000
IN_CH = 512
NUM_RES = 4
HID = 1024
BN = 1000
NSTRIPE = IN_CH // 128
HSTRIPE = NSTRIPE // 2
HCH = IN_CH // 2


def _head_kernel(xa_ref, xb_ref, wl_ref, W1_ref, b1_ref, w2_ref, bias_ref,
                 out_ref):
    fa = xa_ref.reshape(HSTRIPE * NUM_RES * BN, 128)
    fb = xb_ref.reshape(HSTRIPE * NUM_RES * BN, 128)

    def residue(r):
        ra = fa[pl.ds(r, HSTRIPE * BN, NUM_RES), :].reshape(BN, HCH)
        rb = fb[pl.ds(r, HSTRIPE * BN, NUM_RES), :].reshape(BN, HCH)
        return ra, rb

    ra, rb = residue(NUM_RES - 1)
    xlast = jnp.concatenate([ra, rb], axis=1)
    h = jnp.dot(xlast.astype(jnp.bfloat16), W1_ref[...],
                preferred_element_type=jnp.float32)
    h = h + b1_ref[...]
    h = h * jax.nn.sigmoid(h)
    acc = jnp.dot(h, w2_ref[...],
                  preferred_element_type=jnp.float32)
    for r in range(NUM_RES - 1):
        ra, rb = residue(r)
        yr = (jnp.dot(ra, wl_ref[0:HCH, :],
                      preferred_element_type=jnp.float32) +
              jnp.dot(rb, wl_ref[HCH:IN_CH, :],
                      preferred_element_type=jnp.float32))
        acc = acc + yr[:, r:r + 1]
    out_ref[...] = acc + bias_ref[...]


def kernel(embedding_0, W_lin, b_lin, W1, b1, W2, b2):
    x = jnp.squeeze(embedding_0, 2)
    x = x.reshape(N, NSTRIPE, 128, NUM_RES)
    x = jnp.transpose(x, (0, 1, 3, 2))
    wl = jnp.concatenate(
        [W_lin[:, :, 0], jnp.zeros((1, IN_CH), jnp.float32)], axis=0).T
    bias = (jnp.sum(b_lin) + b2[0]).reshape(1, 1)

    def half_spec(t0):
        return pl.BlockSpec((BN, HSTRIPE, NUM_RES, 128),
                            lambda i, t0=t0: (i, t0, 0, 0))

    out = pl.pallas_call(
        _head_kernel,
        grid=(N // BN,),
        in_specs=[
            half_spec(0),
            half_spec(1),
            pl.BlockSpec((IN_CH, NUM_RES), lambda i: (0, 0)),
            pl.BlockSpec((IN_CH, HID), lambda i: (0, 0)),
            pl.BlockSpec((1, HID), lambda i: (0, 0)),
            pl.BlockSpec((HID, 1), lambda i: (0, 0)),
            pl.BlockSpec((1, 1), lambda i: (0, 0)),
        ],
        out_specs=pl.BlockSpec((BN, 1), lambda i: (i, 0)),
        out_shape=jax.ShapeDtypeStruct((N, 1), jnp.float32),
        compiler_params=pltpu.CompilerParams(dimension_semantics=("parallel",)),
    )(x, x, wl, W1.astype(jnp.bfloat16), b1.reshape(1, HID),
      W2.reshape(HID, 1), bias)
    return out.reshape(N)


# speedup vs baseline: 1.1870x; 1.1481x over previous
import jax
import jax.numpy as jnp
from jax.experimental import pallas as pl
from jax.experimental.pallas import tpu as pltpu

N = 10000
IN_CH = 512
NUM_RES = 4
HID = 1024
BN = 1000
NSTRIPE = IN_CH // 128


def _head_kernel(x0_ref, x1_ref, x2_ref, x3_ref, wl_ref, W1_ref, b1_ref,
                 w2_ref, bias_ref, out_ref):
    flat = [p.reshape(BN * NUM_RES, 128)
            for p in (x0_ref, x1_ref, x2_ref, x3_ref)]
    xlast = jnp.concatenate(
        [f[pl.ds(NUM_RES - 1, BN, NUM_RES), :] for f in flat], axis=1)
    h = jnp.dot(xlast.astype(jnp.bfloat16), W1_ref[...],
                preferred_element_type=jnp.float32)
    h = h + b1_ref[...]
    h = h * jax.nn.sigmoid(h)
    acc = jnp.sum(h * w2_ref[...], axis=1, keepdims=True)
    ph = None
    for r in range(NUM_RES - 1):
        for t, f in enumerate(flat):
            xr = f[pl.ds(r, BN, NUM_RES), :]
            wseg = wl_ref[r, t * 128:(t + 1) * 128][None, :]
            pt = xr * wseg
            ph = pt if ph is None else ph + pt
    acc = acc + jnp.sum(ph, axis=1, keepdims=True)
    out_ref[...] = acc + bias_ref[...]


def kernel(embedding_0, W_lin, b_lin, W1, b1, W2, b2):
    x = jnp.squeeze(embedding_0, 2)
    x = x.reshape(N, NSTRIPE, 128, NUM_RES)
    x = jnp.transpose(x, (0, 1, 3, 2))
    wl = jnp.concatenate(
        [W_lin[:, :, 0], jnp.zeros((1, IN_CH), jnp.float32)], axis=0)
    bias = (jnp.sum(b_lin) + b2[0]).reshape(1, 1)

    def stripe_spec(t):
        return pl.BlockSpec((BN, None, NUM_RES, 128),
                            lambda i, t=t: (i, t, 0, 0))

    out = pl.pallas_call(
        _head_kernel,
        grid=(N // BN,),
        in_specs=[stripe_spec(t) for t in range(NSTRIPE)] + [
            pl.BlockSpec((NUM_RES, IN_CH), lambda i: (0, 0)),
            pl.BlockSpec((IN_CH, HID), lambda i: (0, 0)),
            pl.BlockSpec((1, HID), lambda i: (0, 0)),
            pl.BlockSpec((1, HID), lambda i: (0, 0)),
            pl.BlockSpec((1, 1), lambda i: (0, 0)),
        ],
        out_specs=pl.BlockSpec((BN, 1), lambda i: (i, 0)),
        out_shape=jax.ShapeDtypeStruct((N, 1), jnp.float32),
        compiler_params=pltpu.CompilerParams(dimension_semantics=("parallel",)),
    )(x, x, x, x, wl, W1.astype(jnp.bfloat16), b1.reshape(1, HID),
      W2.reshape(1, HID), bias)
    return out.reshape(N)
